# baseline (speedup 1.0000x reference)
import functools

import jax
import jax.numpy as jnp
from jax import lax
from jax.experimental import pallas as pl
from jax.experimental.pallas import tpu as pltpu
from jax.experimental.pallas import tpu_sc as plsc

N_TOKEN = 1000000
CUTOFF = 100000
D_EMBED = 128
D1 = 32
B_TOK = 1024 * 200

NC, NS = 2, 16
NW = NC * NS
BPW = B_TOK // NW
CH = 128
NCHUNK = BPW // CH

_SC_MESH = plsc.VectorSubcoreMesh(core_axis_name="c", subcore_axis_name="s")


@functools.partial(
    pl.kernel,
    out_type=(
        jax.ShapeDtypeStruct((B_TOK, D_EMBED), jnp.float32),
        jax.ShapeDtypeStruct((B_TOK, D1), jnp.float32),
    ),
    mesh=_SC_MESH,
    compiler_params=pltpu.CompilerParams(use_tc_tiling_on_sc=False),
    scratch_types=[
        pltpu.VMEM((CH,), jnp.int32),
        pltpu.VMEM((CH,), jnp.int32),
        pltpu.VMEM((CH, D_EMBED), jnp.float32),
        pltpu.VMEM((CH, D1), jnp.float32),
        pltpu.SemaphoreType.DMA,
        pltpu.SemaphoreType.DMA,
    ],
)
def _sc_gather(idx0_hbm, idx1_hbm, emb0_hbm, emb1_hbm, g0_hbm, g1_hbm,
               idx0_v, idx1_v, rows0_v, rows1_v, sem0, sem1):
    wid = lax.axis_index("s") * NC + lax.axis_index("c")
    base = wid * BPW

    def chunk(i, carry):
        off = base + i * CH
        pltpu.sync_copy(idx0_hbm.at[pl.ds(off, CH)], idx0_v)
        pltpu.sync_copy(idx1_hbm.at[pl.ds(off, CH)], idx1_v)
        cp0 = pltpu.async_copy(emb0_hbm.at[idx0_v], rows0_v, sem0)
        cp1 = pltpu.async_copy(emb1_hbm.at[idx1_v], rows1_v, sem1)
        cp0.wait()
        pltpu.sync_copy(rows0_v, g0_hbm.at[pl.ds(off, CH)])
        cp1.wait()
        pltpu.sync_copy(rows1_v, g1_hbm.at[pl.ds(off, CH)])
        return carry

    lax.fori_loop(0, NCHUNK, chunk, 0)


_TB = 1024


def _tc_body(m_ref, g0_ref, g1_ref, p0_ref, p1_ref, o_ref):
    dn = (((1,), (1,)), ((), ()))
    a = lax.dot_general(g0_ref[...], p0_ref[...], dn,
                        preferred_element_type=jnp.float32)
    b = lax.dot_general(g1_ref[...], p1_ref[...], dn,
                        preferred_element_type=jnp.float32)
    scale = float(D_EMBED) ** 0.5
    o_ref[...] = jnp.where(m_ref[...] < CUTOFF, a, b) * scale


def _tc_project(idx2d, g0, g1, proj0, proj1):
    grid = (B_TOK // _TB,)
    return pl.pallas_call(
        _tc_body,
        grid=grid,
        in_specs=[
            pl.BlockSpec((_TB, 1), lambda i: (i, 0)),
            pl.BlockSpec((_TB, D_EMBED), lambda i: (i, 0)),
            pl.BlockSpec((_TB, D1), lambda i: (i, 0)),
            pl.BlockSpec((D_EMBED, D_EMBED), lambda i: (0, 0)),
            pl.BlockSpec((D_EMBED, D1), lambda i: (0, 0)),
        ],
        out_specs=pl.BlockSpec((_TB, D_EMBED), lambda i: (i, 0)),
        out_shape=jax.ShapeDtypeStruct((B_TOK, D_EMBED), jnp.float32),
    )(idx2d, g0, g1, proj0, proj1)


def kernel(inp, emb0, proj0, emb1, proj1):
    idx = inp.reshape(-1).astype(jnp.int32)
    idx0 = jnp.clip(idx, 0, CUTOFF - 1)
    idx1 = jnp.clip(idx - CUTOFF, 0, N_TOKEN - CUTOFF - 1)
    g0, g1 = _sc_gather(idx0, idx1, emb0, emb1)
    out = _tc_project(idx.reshape(B_TOK, 1), g0, g1, proj0, proj1)
    return out.reshape(inp.shape + (D_EMBED,))
